# baseline (speedup 1.0000x reference)
import functools

import jax
import jax.numpy as jnp
from jax import lax
from jax.experimental import pallas as pl
from jax.experimental.pallas import tpu as pltpu
from jax.experimental.pallas import tpu_sc as plsc

N = 10000
E = 320000
D = 128
G = 64
NC = 2
NS = 16
NW = NC * NS
CH = 128
CPW = 80
CPH = CPW // 2
EPW = CPW * CH
E_PAD = NW * EPW
NT = 10112
RPT = NT // NS

_mesh = plsc.VectorSubcoreMesh(core_axis_name="c", subcore_axis_name="s")


@functools.partial(
    pl.kernel,
    out_type=jax.ShapeDtypeStruct((NC, NT), jnp.float32),
    mesh=_mesh,
    scratch_types=[
        pltpu.VMEM((CPW, CH), jnp.int32),
        pltpu.VMEM((CH,), jnp.float32),
        pltpu.VMEM_SHARED((NT,), jnp.float32),
    ],
)
def _sc_deg(dst3_hbm, zeros1_hbm, deg_out, dst_v, ones_v, deg_sh):
    c = lax.axis_index("c")
    s = lax.axis_index("s")
    w = c * NS + s

    def fill(i, carry):
        ones_v[pl.ds(i * 16, 16)] = jnp.ones((16,), jnp.float32)
        return carry

    lax.fori_loop(0, CH // 16, fill, 0)

    @pl.when(s == 0)
    def _():
        pltpu.sync_copy(zeros1_hbm, deg_sh)

    pltpu.sync_copy(dst3_hbm.at[w], dst_v)
    plsc.subcore_barrier()

    def body(j, carry):
        pltpu.sync_copy(ones_v, deg_sh.at[dst_v.at[j]], add=True)
        return carry

    lax.fori_loop(0, CPW, body, 0)
    plsc.subcore_barrier()

    @pl.when(s == 0)
    def _():
        pltpu.sync_copy(deg_sh, deg_out.at[c])


@functools.partial(
    pl.kernel,
    out_type=jax.ShapeDtypeStruct((NC, NT, D), jnp.float32),
    mesh=_mesh,
    scratch_types=[
        pltpu.VMEM((CPW, CH), jnp.int32),
        pltpu.VMEM((CPW, CH), jnp.int32),
        pltpu.VMEM((CH, D), jnp.float32),
        pltpu.VMEM_SHARED((NT, D), jnp.float32),
        pltpu.SemaphoreType.DMA,
    ],
)
def _sc_scat(gm_hbm, src3_hbm, dst3_hbm, zeros2_hbm, s_out,
             src_v, dst_v, rows_v, s_sh, sem_a):
    c = lax.axis_index("c")
    s = lax.axis_index("s")
    w = c * NS + s
    r0 = s * RPT

    pltpu.sync_copy(zeros2_hbm.at[pl.ds(r0, RPT)], s_sh.at[pl.ds(r0, RPT)])
    pltpu.sync_copy(src3_hbm.at[w], src_v)
    pltpu.sync_copy(dst3_hbm.at[w], dst_v)
    plsc.subcore_barrier()

    def body(j, carry):
        pltpu.async_copy(gm_hbm.at[src_v.at[j]], rows_v, sem_a).wait()
        pltpu.sync_copy(rows_v, s_sh.at[dst_v.at[j]], add=True)
        return carry

    lax.fori_loop(0, CPW, body, 0)
    plsc.subcore_barrier()
    pltpu.sync_copy(s_sh.at[pl.ds(r0, RPT)], s_out.at[c, pl.ds(r0, RPT)])


@functools.partial(
    pl.kernel,
    out_type=jax.ShapeDtypeStruct((NC, CH), jnp.float32),
    mesh=_mesh,
    scratch_types=[
        pltpu.VMEM((NT,), jnp.float32),
        pltpu.VMEM((NT,), jnp.float32),
        pltpu.VMEM((NT,), jnp.int32),
        pltpu.VMEM((EPW,), jnp.int32),
        pltpu.VMEM((EPW,), jnp.int32),
        pltpu.VMEM((CH,), jnp.float32),
        pltpu.VMEM((1, CH), jnp.int32),
        pltpu.VMEM_SHARED((CH,), jnp.float32),
    ],
    compiler_params=pltpu.CompilerParams(needs_layout_passes=False),
)
def _sc_pool(zq_hbm, dq_hbm, batp_hbm, srcp_hbm, dstp_hbm, iota_hbm,
             zeros1_hbm, r_out,
             zq_v, dq_v, bat_v, src_v, dst_v, r_v, io_v, r_sh):
    c = lax.axis_index("c")
    s = lax.axis_index("s")
    w = c * NS + s

    pltpu.sync_copy(zq_hbm.at[0], zq_v)
    pltpu.sync_copy(dq_hbm.at[0], dq_v)
    pltpu.sync_copy(batp_hbm, bat_v)
    pltpu.sync_copy(srcp_hbm.at[pl.ds(w * EPW, EPW)], src_v)
    pltpu.sync_copy(dstp_hbm.at[pl.ds(w * EPW, EPW)], dst_v)
    pltpu.sync_copy(iota_hbm, io_v)

    @pl.when(s == 0)
    def _():
        pltpu.sync_copy(zeros1_hbm.at[pl.ds(0, CH)], r_sh)

    def zero(i, carry):
        r_v[pl.ds(i * 16, 16)] = jnp.zeros((16,), jnp.float32)
        return carry

    lax.fori_loop(0, CH // 16, zero, 0)
    plsc.subcore_barrier()

    def body(e, carry):
        si = src_v[pl.ds(e * 16, 16)]
        di = dst_v[pl.ds(e * 16, 16)]
        val = plsc.load_gather(zq_v, [si]) * plsc.load_gather(dq_v, [di])
        gi = plsc.load_gather(bat_v, [di])
        plsc.addupdate_scatter(r_v, [gi], val)
        return carry

    lax.fori_loop(0, EPW // 16, body, 0)

    pltpu.sync_copy(r_v, r_sh.at[io_v.at[0]], add=True)
    plsc.subcore_barrier()

    @pl.when(s == 0)
    def _():
        pltpu.sync_copy(r_sh, r_out.at[c])


def _tc_prep_body(x_ref, w1_ref, degp_ref, gm_ref, dinv_ref):
    deg = 1.0 + degp_ref[0:1, :] + degp_ref[1:2, :]
    dinv = lax.rsqrt(deg)
    col = lax.broadcasted_iota(jnp.int32, (1, NT), 1)
    dinv = jnp.where(col < N, dinv, 0.0)
    dinv_ref[...] = dinv
    h = jnp.dot(x_ref[...], w1_ref[...], preferred_element_type=jnp.float32)
    dcol = dinv[0, :N].reshape(N, 1)
    gm_ref[...] = h * dcol


def _tc_prep(x, W1, deg_parts):
    return pl.pallas_call(
        _tc_prep_body,
        out_shape=(
            jax.ShapeDtypeStruct((N, D), jnp.float32),
            jax.ShapeDtypeStruct((1, NT), jnp.float32),
        ),
    )(x, W1, deg_parts)


def _tc_mid_body(sp_ref, gm_ref, dinv_ref, b1_ref, w2_ref, wc_ref, zq_ref):
    S = sp_ref[0, 0:N, :] + sp_ref[1, 0:N, :]
    dcol = dinv_ref[0, 0:N].reshape(N, 1)
    h1 = dcol * (S + gm_ref[...]) + b1_ref[0, :][None, :]
    h1r = jnp.maximum(h1, 0.0)
    v = jnp.dot(w2_ref[...], wc_ref[...], preferred_element_type=jnp.float32)
    z = jnp.dot(h1r, v, preferred_element_type=jnp.float32)
    zq = dcol * z
    zq_ref[:, pl.ds(0, N)] = zq.reshape(1, N)
    zq_ref[:, pl.ds(N, NT - N)] = jnp.zeros((1, NT - N), jnp.float32)


def _tc_mid(s_parts, gm, dinv, b1, W2, Wc):
    return pl.pallas_call(
        _tc_mid_body,
        out_shape=jax.ShapeDtypeStruct((1, NT), jnp.float32),
    )(s_parts, gm, dinv, b1, W2, Wc)


def _tc_final_body(rp_ref, zq_ref, dinv_ref, bat_ref, b2_ref, wc_ref, bc_ref,
                   out_ref):
    r_edge = rp_ref[0, 0:G] + rp_ref[1, 0:G]
    zqv = zq_ref[0, 0:N]
    dv = dinv_ref[0, 0:N]
    contrib = zqv * dv
    bat = bat_ref[0, :]
    gid = lax.broadcasted_iota(jnp.int32, (G, N), 0)
    m = (gid == bat[None, :]).astype(jnp.float32)
    r_self = jnp.sum(m * contrib[None, :], axis=1)
    counts = jnp.sum(m, axis=1)
    cm = jnp.maximum(counts, 1.0)
    bw = jnp.dot(b2_ref[...], wc_ref[...],
                 preferred_element_type=jnp.float32)[0, 0]
    outv = (r_edge + r_self) / cm + (counts / cm) * bw + bc_ref[0, 0]
    out_ref[...] = outv.reshape(G, 1)


def _tc_final(r_parts, zq, dinv, batch2, b2, Wc, bc):
    return pl.pallas_call(
        _tc_final_body,
        out_shape=jax.ShapeDtypeStruct((G, 1), jnp.float32),
    )(r_parts, zq, dinv, batch2, b2, Wc, bc)


def kernel(x, edge_index, batch, W1, b1, W2, b2, Wc, bc):
    src = edge_index[0].astype(jnp.int32)
    dst = edge_index[1].astype(jnp.int32)
    bat = batch.astype(jnp.int32)

    pad = E_PAD - E
    src_p = jnp.concatenate([src, jnp.zeros((pad,), jnp.int32)])
    dst_p = jnp.concatenate([dst, jnp.full((pad,), N, jnp.int32)])
    src3 = src_p.reshape(NW, CPW, CH)
    dst3 = dst_p.reshape(NW, CPW, CH)
    src4 = src_p.reshape(NW, CPW // 2, 2, CH)
    dst4 = dst_p.reshape(NW, CPW // 2, 2, CH)
    batp = jnp.concatenate([bat, jnp.full((NT - N,), G, jnp.int32)])
    zeros1 = jnp.zeros((NT,), jnp.float32)
    zeros2 = jnp.zeros((NT, D), jnp.float32)
    iota = jnp.arange(CH, dtype=jnp.int32).reshape(1, CH)

    deg_parts = _sc_deg(dst3, zeros1)
    gm, dinv = _tc_prep(x, W1, deg_parts)
    s_parts = _sc_scat(gm, src3, dst3, zeros2)
    zq = _tc_mid(s_parts, gm, dinv, b1.reshape(1, D), W2, Wc)
    r_parts = _sc_pool(zq, dinv, batp, src_p, dst_p, iota, zeros1)
    out = _tc_final(r_parts, zq, dinv, bat.reshape(1, N), b2.reshape(1, D),
                    Wc, bc.reshape(1, 1))
    return out


# speedup vs baseline: 2.1821x; 2.1821x over previous
import functools

import jax
import jax.numpy as jnp
from jax import lax
from jax.experimental import pallas as pl
from jax.experimental.pallas import tpu as pltpu
from jax.experimental.pallas import tpu_sc as plsc

N = 10000
E = 320000
D = 128
G = 64
NC = 2
NS = 16
NW = NC * NS
CH = 128
CPW = 80
CPH = CPW // 2
EPW = CPW * CH
E_PAD = NW * EPW
NT = 10112
RPT = NT // NS

_mesh = plsc.VectorSubcoreMesh(core_axis_name="c", subcore_axis_name="s")


@functools.partial(
    pl.kernel,
    out_type=jax.ShapeDtypeStruct((NC, NT), jnp.float32),
    mesh=_mesh,
    scratch_types=[
        pltpu.VMEM((CPW, CH), jnp.int32),
        pltpu.VMEM((CH,), jnp.float32),
        pltpu.VMEM_SHARED((NT,), jnp.float32),
    ],
)
def _sc_deg(dst3_hbm, zeros1_hbm, deg_out, dst_v, ones_v, deg_sh):
    c = lax.axis_index("c")
    s = lax.axis_index("s")
    w = c * NS + s

    def fill(i, carry):
        ones_v[pl.ds(i * 16, 16)] = jnp.ones((16,), jnp.float32)
        return carry

    lax.fori_loop(0, CH // 16, fill, 0)

    @pl.when(s == 0)
    def _():
        pltpu.sync_copy(zeros1_hbm, deg_sh)

    pltpu.sync_copy(dst3_hbm.at[w], dst_v)
    plsc.subcore_barrier()

    def body(j, carry):
        pltpu.sync_copy(ones_v, deg_sh.at[dst_v.at[j]], add=True)
        return carry

    lax.fori_loop(0, CPW, body, 0)
    plsc.subcore_barrier()

    @pl.when(s == 0)
    def _():
        pltpu.sync_copy(deg_sh, deg_out.at[c])


@functools.partial(
    pl.kernel,
    out_type=jax.ShapeDtypeStruct((NC, NT, D), jnp.float32),
    mesh=_mesh,
    scratch_types=[
        pltpu.VMEM((CPW, CH), jnp.int32),
        pltpu.VMEM((CPW, CH), jnp.int32),
        pltpu.VMEM((CH, D), jnp.float32),
        pltpu.VMEM_SHARED((NT, D), jnp.float32),
        pltpu.SemaphoreType.DMA,
    ],
)
def _sc_scat(gm_hbm, src3_hbm, dst3_hbm, zeros2_hbm, s_out,
             src_v, dst_v, rows_v, s_sh, sem_a):
    c = lax.axis_index("c")
    s = lax.axis_index("s")
    w = c * NS + s
    r0 = s * RPT

    pltpu.sync_copy(zeros2_hbm.at[pl.ds(r0, RPT)], s_sh.at[pl.ds(r0, RPT)])
    pltpu.sync_copy(src3_hbm.at[w], src_v)
    pltpu.sync_copy(dst3_hbm.at[w], dst_v)
    plsc.subcore_barrier()

    def body(j, carry):
        pltpu.async_copy(gm_hbm.at[src_v.at[j]], rows_v, sem_a).wait()
        pltpu.sync_copy(rows_v, s_sh.at[dst_v.at[j]], add=True)
        return carry

    lax.fori_loop(0, CPW, body, 0)
    plsc.subcore_barrier()
    pltpu.sync_copy(s_sh.at[pl.ds(r0, RPT)], s_out.at[c, pl.ds(r0, RPT)])


@functools.partial(
    pl.kernel,
    out_type=jax.ShapeDtypeStruct((NC, CH), jnp.float32),
    mesh=_mesh,
    scratch_types=[
        pltpu.VMEM((NT,), jnp.float32),
        pltpu.VMEM((NT,), jnp.float32),
        pltpu.VMEM((NT,), jnp.int32),
        pltpu.VMEM((EPW,), jnp.int32),
        pltpu.VMEM((EPW,), jnp.int32),
        pltpu.VMEM((CH,), jnp.float32),
        pltpu.VMEM((1, CH), jnp.int32),
        pltpu.VMEM_SHARED((CH,), jnp.float32),
    ],
    compiler_params=pltpu.CompilerParams(needs_layout_passes=False),
)
def _sc_pool(zq_hbm, dq_hbm, batp_hbm, srcp_hbm, dstp_hbm, iota_hbm,
             zeros1_hbm, r_out,
             zq_v, dq_v, bat_v, src_v, dst_v, r_v, io_v, r_sh):
    c = lax.axis_index("c")
    s = lax.axis_index("s")
    w = c * NS + s

    pltpu.sync_copy(zq_hbm.at[0], zq_v)
    pltpu.sync_copy(dq_hbm.at[0], dq_v)
    pltpu.sync_copy(batp_hbm, bat_v)
    pltpu.sync_copy(srcp_hbm.at[pl.ds(w * EPW, EPW)], src_v)
    pltpu.sync_copy(dstp_hbm.at[pl.ds(w * EPW, EPW)], dst_v)
    pltpu.sync_copy(iota_hbm, io_v)

    @pl.when(s == 0)
    def _():
        pltpu.sync_copy(zeros1_hbm.at[pl.ds(0, CH)], r_sh)

    def zero(i, carry):
        r_v[pl.ds(i * 16, 16)] = jnp.zeros((16,), jnp.float32)
        return carry

    lax.fori_loop(0, CH // 16, zero, 0)
    plsc.subcore_barrier()

    def body(e, carry):
        si = src_v[pl.ds(e * 16, 16)]
        di = dst_v[pl.ds(e * 16, 16)]
        val = plsc.load_gather(zq_v, [si]) * plsc.load_gather(dq_v, [di])
        gi = plsc.load_gather(bat_v, [di])
        plsc.addupdate_scatter(r_v, [gi], val)
        return carry

    lax.fori_loop(0, EPW // 16, body, 0)

    pltpu.sync_copy(r_v, r_sh.at[io_v.at[0]], add=True)
    plsc.subcore_barrier()

    @pl.when(s == 0)
    def _():
        pltpu.sync_copy(r_sh, r_out.at[c])


def _tc_prep_body(x_ref, w1_ref, degp_ref, gm_ref, dinv_ref):
    deg = 1.0 + degp_ref[0:1, :] + degp_ref[1:2, :]
    dinv = lax.rsqrt(deg)
    col = lax.broadcasted_iota(jnp.int32, (1, NT), 1)
    dinv = jnp.where(col < N, dinv, 0.0)
    dinv_ref[...] = dinv
    h = jnp.dot(x_ref[...], w1_ref[...], preferred_element_type=jnp.float32)
    dcol = dinv[0, :N].reshape(N, 1)
    gm_ref[...] = h * dcol


def _tc_prep(x, W1, deg_parts):
    return pl.pallas_call(
        _tc_prep_body,
        out_shape=(
            jax.ShapeDtypeStruct((N, D), jnp.float32),
            jax.ShapeDtypeStruct((1, NT), jnp.float32),
        ),
    )(x, W1, deg_parts)


def _tc_mid_body(sp_ref, gm_ref, dinv_ref, b1_ref, w2_ref, wc_ref, zq_ref):
    S = sp_ref[0, 0:N, :] + sp_ref[1, 0:N, :]
    dcol = dinv_ref[0, 0:N].reshape(N, 1)
    h1 = dcol * (S + gm_ref[...]) + b1_ref[0, :][None, :]
    h1r = jnp.maximum(h1, 0.0)
    v = jnp.dot(w2_ref[...], wc_ref[...], preferred_element_type=jnp.float32)
    z = jnp.dot(h1r, v, preferred_element_type=jnp.float32)
    zq = dcol * z
    zq_ref[:, pl.ds(0, N)] = zq.reshape(1, N)
    zq_ref[:, pl.ds(N, NT - N)] = jnp.zeros((1, NT - N), jnp.float32)


def _tc_mid(s_parts, gm, dinv, b1, W2, Wc):
    return pl.pallas_call(
        _tc_mid_body,
        out_shape=jax.ShapeDtypeStruct((1, NT), jnp.float32),
    )(s_parts, gm, dinv, b1, W2, Wc)


def _tc_final_body(rp_ref, zq_ref, dinv_ref, bat_ref, b2_ref, wc_ref, bc_ref,
                   out_ref):
    r_edge = rp_ref[0, 0:G] + rp_ref[1, 0:G]
    zqv = zq_ref[0, 0:N]
    dv = dinv_ref[0, 0:N]
    contrib = zqv * dv
    bat = bat_ref[0, :]
    gid = lax.broadcasted_iota(jnp.int32, (G, N), 0)
    m = (gid == bat[None, :]).astype(jnp.float32)
    r_self = jnp.sum(m * contrib[None, :], axis=1)
    counts = jnp.sum(m, axis=1)
    cm = jnp.maximum(counts, 1.0)
    bw = jnp.dot(b2_ref[...], wc_ref[...],
                 preferred_element_type=jnp.float32)[0, 0]
    outv = (r_edge + r_self) / cm + (counts / cm) * bw + bc_ref[0, 0]
    out_ref[...] = outv.reshape(G, 1)


def _tc_final(r_parts, zq, dinv, batch2, b2, Wc, bc):
    return pl.pallas_call(
        _tc_final_body,
        out_shape=jax.ShapeDtypeStruct((G, 1), jnp.float32),
    )(r_parts, zq, dinv, batch2, b2, Wc, bc)


def kernel(x, edge_index, batch, W1, b1, W2, b2, Wc, bc):
    src = edge_index[0].astype(jnp.int32)
    dst = edge_index[1].astype(jnp.int32)
    bat = batch.astype(jnp.int32)

    pad = E_PAD - E
    pad_i = jnp.arange(pad, dtype=jnp.int32)
    src_p = jnp.concatenate([src, pad_i % N])
    dst_p = jnp.concatenate([dst, N + pad_i % (NT - N)])
    src3 = src_p.reshape(NW, CPW, CH)
    dst3 = dst_p.reshape(NW, CPW, CH)
    src4 = src_p.reshape(NW, CPW // 2, 2, CH)
    dst4 = dst_p.reshape(NW, CPW // 2, 2, CH)
    batp = jnp.concatenate([bat, jnp.full((NT - N,), G, jnp.int32)])
    zeros1 = jnp.zeros((NT,), jnp.float32)
    zeros2 = jnp.zeros((NT, D), jnp.float32)
    iota = jnp.arange(CH, dtype=jnp.int32).reshape(1, CH)

    deg_parts = _sc_deg(dst3, zeros1)
    gm, dinv = _tc_prep(x, W1, deg_parts)
    s_parts = _sc_scat(gm, src3, dst3, zeros2)
    zq = _tc_mid(s_parts, gm, dinv, b1.reshape(1, D), W2, Wc)
    r_parts = _sc_pool(zq, dinv, batp, src_p, dst_p, iota, zeros1)
    out = _tc_final(r_parts, zq, dinv, bat.reshape(1, N), b2.reshape(1, D),
                    Wc, bc.reshape(1, 1))
    return out


# speedup vs baseline: 2.8246x; 1.2944x over previous
import functools

import jax
import jax.numpy as jnp
from jax import lax
from jax.experimental import pallas as pl
from jax.experimental.pallas import tpu as pltpu
from jax.experimental.pallas import tpu_sc as plsc

N = 10000
E = 320000
D = 128
G = 64
NC = 2
NS = 16
NW = NC * NS
CH = 128
CPW = 80
CPH = CPW // 2
EPW = CPW * CH
E_PAD = NW * EPW
NT = 10112
RPT = NT // NS

_mesh = plsc.VectorSubcoreMesh(core_axis_name="c", subcore_axis_name="s")


@functools.partial(
    pl.kernel,
    out_type=jax.ShapeDtypeStruct((NC, NT), jnp.float32),
    mesh=_mesh,
    scratch_types=[
        pltpu.VMEM((CPW, CH), jnp.int32),
        pltpu.VMEM((CH,), jnp.float32),
        pltpu.VMEM_SHARED((NT,), jnp.float32),
    ],
)
def _sc_deg(dst3_hbm, zeros1_hbm, deg_out, dst_v, ones_v, deg_sh):
    c = lax.axis_index("c")
    s = lax.axis_index("s")
    w = c * NS + s

    def fill(i, carry):
        ones_v[pl.ds(i * 16, 16)] = jnp.ones((16,), jnp.float32)
        return carry

    lax.fori_loop(0, CH // 16, fill, 0)

    @pl.when(s == 0)
    def _():
        pltpu.sync_copy(zeros1_hbm, deg_sh)

    pltpu.sync_copy(dst3_hbm.at[w], dst_v)
    plsc.subcore_barrier()

    def body(j, carry):
        pltpu.sync_copy(ones_v, deg_sh.at[dst_v.at[j]], add=True)
        return carry

    lax.fori_loop(0, CPW, body, 0)
    plsc.subcore_barrier()

    @pl.when(s == 0)
    def _():
        pltpu.sync_copy(deg_sh, deg_out.at[c])


@functools.partial(
    pl.kernel,
    out_type=jax.ShapeDtypeStruct((NC, NT, D), jnp.float32),
    mesh=_mesh,
    scratch_types=[
        pltpu.VMEM((CPH, CH), jnp.int32),
        pltpu.VMEM((CPH, CH), jnp.int32),
        pltpu.VMEM((2, CH, D), jnp.float32),
        pltpu.VMEM_SHARED((NT, D), jnp.float32),
        pltpu.SemaphoreType.DMA,
        pltpu.SemaphoreType.DMA,
    ],
)
def _sc_scat(gm_hbm, src3_hbm, dst3_hbm, zeros2_hbm, s_out,
             src_v, dst_v, rows_v, s_sh, sem_a, sem_b):
    c = lax.axis_index("c")
    s = lax.axis_index("s")
    w = c * NS + s
    r0 = s * RPT

    pltpu.sync_copy(zeros2_hbm.at[pl.ds(r0, RPT)], s_sh.at[pl.ds(r0, RPT)])
    plsc.subcore_barrier()

    def gather(j, buf, sem):
        pltpu.async_copy(gm_hbm.at[src_v.at[j]], rows_v.at[buf], sem)

    def gwait(buf, sem):
        pltpu.make_async_copy(gm_hbm.at[src_v.at[0]], rows_v.at[buf], sem).wait()

    def scat(j, buf):
        pltpu.sync_copy(rows_v.at[buf], s_sh.at[dst_v.at[j]], add=True)

    for h in range(2):
        pltpu.sync_copy(src3_hbm.at[w, pl.ds(h * CPH, CPH)], src_v)
        pltpu.sync_copy(dst3_hbm.at[w, pl.ds(h * CPH, CPH)], dst_v)
        gather(0, 0, sem_a)

        def body(k, carry):
            j0 = 2 * k
            gwait(0, sem_a)
            gather(j0 + 1, 1, sem_b)
            scat(j0, 0)

            @pl.when(j0 + 2 < CPH)
            def _():
                gather(j0 + 2, 0, sem_a)

            gwait(1, sem_b)
            scat(j0 + 1, 1)
            return carry

        lax.fori_loop(0, CPH // 2, body, 0)
    plsc.subcore_barrier()
    pltpu.sync_copy(s_sh.at[pl.ds(r0, RPT)], s_out.at[c, pl.ds(r0, RPT)])


@functools.partial(
    pl.kernel,
    out_type=jax.ShapeDtypeStruct((NC, CH), jnp.float32),
    mesh=_mesh,
    scratch_types=[
        pltpu.VMEM((NT,), jnp.float32),
        pltpu.VMEM((NT,), jnp.float32),
        pltpu.VMEM((NT,), jnp.int32),
        pltpu.VMEM((EPW,), jnp.int32),
        pltpu.VMEM((EPW,), jnp.int32),
        pltpu.VMEM((CH,), jnp.float32),
        pltpu.VMEM((1, CH), jnp.int32),
        pltpu.VMEM_SHARED((CH,), jnp.float32),
    ],
    compiler_params=pltpu.CompilerParams(needs_layout_passes=False),
)
def _sc_pool(zq_hbm, dq_hbm, batp_hbm, srcp_hbm, dstp_hbm, iota_hbm,
             zeros1_hbm, r_out,
             zq_v, dq_v, bat_v, src_v, dst_v, r_v, io_v, r_sh):
    c = lax.axis_index("c")
    s = lax.axis_index("s")
    w = c * NS + s

    pltpu.sync_copy(zq_hbm.at[0], zq_v)
    pltpu.sync_copy(dq_hbm.at[0], dq_v)
    pltpu.sync_copy(batp_hbm, bat_v)
    pltpu.sync_copy(srcp_hbm.at[pl.ds(w * EPW, EPW)], src_v)
    pltpu.sync_copy(dstp_hbm.at[pl.ds(w * EPW, EPW)], dst_v)
    pltpu.sync_copy(iota_hbm, io_v)

    @pl.when(s == 0)
    def _():
        pltpu.sync_copy(zeros1_hbm.at[pl.ds(0, CH)], r_sh)

    def zero(i, carry):
        r_v[pl.ds(i * 16, 16)] = jnp.zeros((16,), jnp.float32)
        return carry

    lax.fori_loop(0, CH // 16, zero, 0)
    plsc.subcore_barrier()

    def body(e, carry):
        si = src_v[pl.ds(e * 16, 16)]
        di = dst_v[pl.ds(e * 16, 16)]
        val = plsc.load_gather(zq_v, [si]) * plsc.load_gather(dq_v, [di])
        gi = plsc.load_gather(bat_v, [di])
        plsc.addupdate_scatter(r_v, [gi], val)
        return carry

    lax.fori_loop(0, EPW // 16, body, 0)

    pltpu.sync_copy(r_v, r_sh.at[io_v.at[0]], add=True)
    plsc.subcore_barrier()

    @pl.when(s == 0)
    def _():
        pltpu.sync_copy(r_sh, r_out.at[c])


def _tc_prep_body(x_ref, w1_ref, degp_ref, gm_ref, dinv_ref):
    deg = 1.0 + degp_ref[0:1, :] + degp_ref[1:2, :]
    dinv = lax.rsqrt(deg)
    col = lax.broadcasted_iota(jnp.int32, (1, NT), 1)
    dinv = jnp.where(col < N, dinv, 0.0)
    dinv_ref[...] = dinv
    h = jnp.dot(x_ref[...], w1_ref[...], preferred_element_type=jnp.float32)
    dcol = dinv[0, :N].reshape(N, 1)
    gm_ref[...] = h * dcol


def _tc_prep(x, W1, deg_parts):
    return pl.pallas_call(
        _tc_prep_body,
        out_shape=(
            jax.ShapeDtypeStruct((N, D), jnp.float32),
            jax.ShapeDtypeStruct((1, NT), jnp.float32),
        ),
    )(x, W1, deg_parts)


def _tc_mid_body(sp_ref, gm_ref, dinv_ref, b1_ref, w2_ref, wc_ref, zq_ref):
    S = sp_ref[0, 0:N, :] + sp_ref[1, 0:N, :]
    dcol = dinv_ref[0, 0:N].reshape(N, 1)
    h1 = dcol * (S + gm_ref[...]) + b1_ref[0, :][None, :]
    h1r = jnp.maximum(h1, 0.0)
    v = jnp.dot(w2_ref[...], wc_ref[...], preferred_element_type=jnp.float32)
    z = jnp.dot(h1r, v, preferred_element_type=jnp.float32)
    zq = dcol * z
    zq_ref[:, pl.ds(0, N)] = zq.reshape(1, N)
    zq_ref[:, pl.ds(N, NT - N)] = jnp.zeros((1, NT - N), jnp.float32)


def _tc_mid(s_parts, gm, dinv, b1, W2, Wc):
    return pl.pallas_call(
        _tc_mid_body,
        out_shape=jax.ShapeDtypeStruct((1, NT), jnp.float32),
    )(s_parts, gm, dinv, b1, W2, Wc)


def _tc_final_body(rp_ref, zq_ref, dinv_ref, bat_ref, b2_ref, wc_ref, bc_ref,
                   out_ref):
    r_edge = rp_ref[0, 0:G] + rp_ref[1, 0:G]
    zqv = zq_ref[0, 0:N]
    dv = dinv_ref[0, 0:N]
    contrib = zqv * dv
    bat = bat_ref[0, :]
    gid = lax.broadcasted_iota(jnp.int32, (G, N), 0)
    m = (gid == bat[None, :]).astype(jnp.float32)
    r_self = jnp.sum(m * contrib[None, :], axis=1)
    counts = jnp.sum(m, axis=1)
    cm = jnp.maximum(counts, 1.0)
    bw = jnp.dot(b2_ref[...], wc_ref[...],
                 preferred_element_type=jnp.float32)[0, 0]
    outv = (r_edge + r_self) / cm + (counts / cm) * bw + bc_ref[0, 0]
    out_ref[...] = outv.reshape(G, 1)


def _tc_final(r_parts, zq, dinv, batch2, b2, Wc, bc):
    return pl.pallas_call(
        _tc_final_body,
        out_shape=jax.ShapeDtypeStruct((G, 1), jnp.float32),
    )(r_parts, zq, dinv, batch2, b2, Wc, bc)


def kernel(x, edge_index, batch, W1, b1, W2, b2, Wc, bc):
    src = edge_index[0].astype(jnp.int32)
    dst = edge_index[1].astype(jnp.int32)
    bat = batch.astype(jnp.int32)

    pad = E_PAD - E
    pad_i = jnp.arange(pad, dtype=jnp.int32)
    src_p = jnp.concatenate([src, pad_i % N])
    dst_p = jnp.concatenate([dst, N + pad_i % (NT - N)])
    src3 = src_p.reshape(NW, CPW, CH)
    dst3 = dst_p.reshape(NW, CPW, CH)
    src4 = src_p.reshape(NW, CPW // 2, 2, CH)
    dst4 = dst_p.reshape(NW, CPW // 2, 2, CH)
    batp = jnp.concatenate([bat, jnp.full((NT - N,), G, jnp.int32)])
    zeros1 = jnp.zeros((NT,), jnp.float32)
    zeros2 = jnp.zeros((NT, D), jnp.float32)
    iota = jnp.arange(CH, dtype=jnp.int32).reshape(1, CH)

    deg_parts = _sc_deg(dst3, zeros1)
    gm, dinv = _tc_prep(x, W1, deg_parts)
    s_parts = _sc_scat(gm, src3, dst3, zeros2)
    zq = _tc_mid(s_parts, gm, dinv, b1.reshape(1, D), W2, Wc)
    r_parts = _sc_pool(zq, dinv, batp, src_p, dst_p, iota, zeros1)
    out = _tc_final(r_parts, zq, dinv, bat.reshape(1, N), b2.reshape(1, D),
                    Wc, bc.reshape(1, 1))
    return out


# speedup vs baseline: 2.8593x; 1.0123x over previous
import functools

import jax
import jax.numpy as jnp
from jax import lax
from jax.experimental import pallas as pl
from jax.experimental.pallas import tpu as pltpu
from jax.experimental.pallas import tpu_sc as plsc

N = 10000
E = 320000
D = 128
G = 64
NC = 2
NS = 16
NW = NC * NS
CH = 128
CPW = 80
CPH = CPW // 2
EPW = CPW * CH
E_PAD = NW * EPW
NT = 10112
RPT = NT // NS

_mesh = plsc.VectorSubcoreMesh(core_axis_name="c", subcore_axis_name="s")


@functools.partial(
    pl.kernel,
    out_type=jax.ShapeDtypeStruct((NC, NT), jnp.float32),
    mesh=_mesh,
    scratch_types=[
        pltpu.VMEM((CPW, CH), jnp.int32),
        pltpu.VMEM((CH,), jnp.float32),
        pltpu.VMEM_SHARED((NT,), jnp.float32),
    ],
)
def _sc_deg(dst3_hbm, zeros1_hbm, deg_out, dst_v, ones_v, deg_sh):
    c = lax.axis_index("c")
    s = lax.axis_index("s")
    w = c * NS + s

    def fill(i, carry):
        ones_v[pl.ds(i * 16, 16)] = jnp.ones((16,), jnp.float32)
        return carry

    lax.fori_loop(0, CH // 16, fill, 0)

    @pl.when(s == 0)
    def _():
        pltpu.sync_copy(zeros1_hbm, deg_sh)

    pltpu.sync_copy(dst3_hbm.at[w], dst_v)
    plsc.subcore_barrier()

    def body(j, carry):
        pltpu.sync_copy(ones_v, deg_sh.at[dst_v.at[j]], add=True)
        return carry

    lax.fori_loop(0, CPW, body, 0)
    plsc.subcore_barrier()

    @pl.when(s == 0)
    def _():
        pltpu.sync_copy(deg_sh, deg_out.at[c])


@functools.partial(
    pl.kernel,
    out_type=jax.ShapeDtypeStruct((NC, NT, D), jnp.float32),
    mesh=_mesh,
    scratch_types=[
        pltpu.VMEM((CPH, CH), jnp.int32),
        pltpu.VMEM((CPH, CH), jnp.int32),
        pltpu.VMEM((2, CH, D), jnp.float32),
        pltpu.VMEM_SHARED((NT, D), jnp.float32),
        pltpu.SemaphoreType.DMA,
        pltpu.SemaphoreType.DMA,
    ],
)
def _sc_scat(gm_hbm, src3_hbm, dst3_hbm, zeros2_hbm, s_out,
             src_v, dst_v, rows_v, s_sh, sem_a, sem_b):
    c = lax.axis_index("c")
    s = lax.axis_index("s")
    w = c * NS + s
    r0 = s * RPT

    pltpu.sync_copy(zeros2_hbm.at[pl.ds(r0, RPT)], s_sh.at[pl.ds(r0, RPT)])
    plsc.subcore_barrier()

    def gather(j, buf, sem):
        pltpu.async_copy(gm_hbm.at[src_v.at[j]], rows_v.at[buf], sem)

    def gwait(buf, sem):
        pltpu.make_async_copy(gm_hbm.at[src_v.at[0]], rows_v.at[buf], sem).wait()

    def scat(j, buf):
        pltpu.sync_copy(rows_v.at[buf], s_sh.at[dst_v.at[j]], add=True)

    for h in range(2):
        pltpu.sync_copy(src3_hbm.at[w, pl.ds(h * CPH, CPH)], src_v)
        pltpu.sync_copy(dst3_hbm.at[w, pl.ds(h * CPH, CPH)], dst_v)
        gather(0, 0, sem_a)

        def body(k, carry):
            j0 = 2 * k
            gwait(0, sem_a)
            gather(j0 + 1, 1, sem_b)
            scat(j0, 0)

            @pl.when(j0 + 2 < CPH)
            def _():
                gather(j0 + 2, 0, sem_a)

            gwait(1, sem_b)
            scat(j0 + 1, 1)
            return carry

        lax.fori_loop(0, CPH // 2, body, 0)
    plsc.subcore_barrier()
    pltpu.sync_copy(s_sh.at[pl.ds(r0, RPT)], s_out.at[c, pl.ds(r0, RPT)])


@functools.partial(
    pl.kernel,
    out_type=jax.ShapeDtypeStruct((NC, CH), jnp.float32),
    mesh=_mesh,
    scratch_types=[
        pltpu.VMEM((NT,), jnp.float32),
        pltpu.VMEM((NT,), jnp.float32),
        pltpu.VMEM((NT,), jnp.int32),
        pltpu.VMEM((EPW,), jnp.int32),
        pltpu.VMEM((EPW,), jnp.int32),
        pltpu.VMEM((16 * CH,), jnp.float32),
        pltpu.VMEM((CH,), jnp.float32),
        pltpu.VMEM((1, CH), jnp.int32),
        pltpu.VMEM_SHARED((CH,), jnp.float32),
    ],
    compiler_params=pltpu.CompilerParams(needs_layout_passes=False),
)
def _sc_pool(zq_hbm, dq_hbm, batp_hbm, srcp_hbm, dstp_hbm, iota_hbm,
             zeros1_hbm, r_out,
             zq_v, dq_v, bat_v, src_v, dst_v, r_v, r2_v, io_v, r_sh):
    c = lax.axis_index("c")
    s = lax.axis_index("s")
    w = c * NS + s

    pltpu.sync_copy(zq_hbm.at[0], zq_v)
    pltpu.sync_copy(dq_hbm.at[0], dq_v)
    pltpu.sync_copy(batp_hbm, bat_v)
    pltpu.sync_copy(srcp_hbm.at[pl.ds(w * EPW, EPW)], src_v)
    pltpu.sync_copy(dstp_hbm.at[pl.ds(w * EPW, EPW)], dst_v)
    pltpu.sync_copy(iota_hbm, io_v)

    @pl.when(s == 0)
    def _():
        pltpu.sync_copy(zeros1_hbm.at[pl.ds(0, CH)], r_sh)

    def zero(i, carry):
        r_v[pl.ds(i * 16, 16)] = jnp.zeros((16,), jnp.float32)
        return carry

    lax.fori_loop(0, 16 * CH // 16, zero, 0)
    plsc.subcore_barrier()

    lane_off = jnp.arange(16, dtype=jnp.int32) * CH

    def body(e, carry):
        si = src_v[pl.ds(e * 16, 16)]
        di = dst_v[pl.ds(e * 16, 16)]
        val = plsc.load_gather(zq_v, [si]) * plsc.load_gather(dq_v, [di])
        gi = plsc.load_gather(bat_v, [di])
        plsc.addupdate_scatter(r_v, [gi + lane_off], val)
        return carry

    lax.fori_loop(0, EPW // 16, body, 0)

    for cc in range(CH // 16):
        acc = jnp.zeros((16,), jnp.float32)
        for l in range(16):
            acc = acc + r_v[pl.ds(l * CH + cc * 16, 16)]
        r2_v[pl.ds(cc * 16, 16)] = acc

    pltpu.sync_copy(r2_v, r_sh.at[io_v.at[0]], add=True)
    plsc.subcore_barrier()

    @pl.when(s == 0)
    def _():
        pltpu.sync_copy(r_sh, r_out.at[c])


def _tc_prep_body(x_ref, w1_ref, degp_ref, gm_ref, dinv_ref):
    deg = 1.0 + degp_ref[0:1, :] + degp_ref[1:2, :]
    dinv = lax.rsqrt(deg)
    col = lax.broadcasted_iota(jnp.int32, (1, NT), 1)
    dinv = jnp.where(col < N, dinv, 0.0)
    dinv_ref[...] = dinv
    h = jnp.dot(x_ref[...], w1_ref[...], preferred_element_type=jnp.float32)
    dcol = dinv[0, :N].reshape(N, 1)
    gm_ref[...] = h * dcol


def _tc_prep(x, W1, deg_parts):
    return pl.pallas_call(
        _tc_prep_body,
        out_shape=(
            jax.ShapeDtypeStruct((N, D), jnp.float32),
            jax.ShapeDtypeStruct((1, NT), jnp.float32),
        ),
    )(x, W1, deg_parts)


def _tc_mid_body(sp_ref, gm_ref, dinv_ref, b1_ref, w2_ref, wc_ref, zq_ref):
    S = sp_ref[0, 0:N, :] + sp_ref[1, 0:N, :]
    dcol = dinv_ref[0, 0:N].reshape(N, 1)
    h1 = dcol * (S + gm_ref[...]) + b1_ref[0, :][None, :]
    h1r = jnp.maximum(h1, 0.0)
    v = jnp.dot(w2_ref[...], wc_ref[...], preferred_element_type=jnp.float32)
    z = jnp.dot(h1r, v, preferred_element_type=jnp.float32)
    zq = dcol * z
    zq_ref[:, pl.ds(0, N)] = zq.reshape(1, N)
    zq_ref[:, pl.ds(N, NT - N)] = jnp.zeros((1, NT - N), jnp.float32)


def _tc_mid(s_parts, gm, dinv, b1, W2, Wc):
    return pl.pallas_call(
        _tc_mid_body,
        out_shape=jax.ShapeDtypeStruct((1, NT), jnp.float32),
    )(s_parts, gm, dinv, b1, W2, Wc)


def _tc_final_body(rp_ref, zq_ref, dinv_ref, bat_ref, b2_ref, wc_ref, bc_ref,
                   out_ref):
    r_edge = rp_ref[0, 0:G] + rp_ref[1, 0:G]
    zqv = zq_ref[0, 0:N]
    dv = dinv_ref[0, 0:N]
    contrib = zqv * dv
    bat = bat_ref[0, :]
    gid = lax.broadcasted_iota(jnp.int32, (G, N), 0)
    m = (gid == bat[None, :]).astype(jnp.float32)
    r_self = jnp.sum(m * contrib[None, :], axis=1)
    counts = jnp.sum(m, axis=1)
    cm = jnp.maximum(counts, 1.0)
    bw = jnp.dot(b2_ref[...], wc_ref[...],
                 preferred_element_type=jnp.float32)[0, 0]
    outv = (r_edge + r_self) / cm + (counts / cm) * bw + bc_ref[0, 0]
    out_ref[...] = outv.reshape(G, 1)


def _tc_final(r_parts, zq, dinv, batch2, b2, Wc, bc):
    return pl.pallas_call(
        _tc_final_body,
        out_shape=jax.ShapeDtypeStruct((G, 1), jnp.float32),
    )(r_parts, zq, dinv, batch2, b2, Wc, bc)


def kernel(x, edge_index, batch, W1, b1, W2, b2, Wc, bc):
    src = edge_index[0].astype(jnp.int32)
    dst = edge_index[1].astype(jnp.int32)
    bat = batch.astype(jnp.int32)

    pad = E_PAD - E
    pad_i = jnp.arange(pad, dtype=jnp.int32)
    src_p = jnp.concatenate([src, pad_i % N])
    dst_p = jnp.concatenate([dst, N + pad_i % (NT - N)])
    src3 = src_p.reshape(NW, CPW, CH)
    dst3 = dst_p.reshape(NW, CPW, CH)
    src4 = src_p.reshape(NW, CPW // 2, 2, CH)
    dst4 = dst_p.reshape(NW, CPW // 2, 2, CH)
    batp = jnp.concatenate([bat, jnp.full((NT - N,), G, jnp.int32)])
    zeros1 = jnp.zeros((NT,), jnp.float32)
    zeros2 = jnp.zeros((NT, D), jnp.float32)
    iota = jnp.arange(CH, dtype=jnp.int32).reshape(1, CH)

    deg_parts = _sc_deg(dst3, zeros1)
    gm, dinv = _tc_prep(x, W1, deg_parts)
    s_parts = _sc_scat(gm, src3, dst3, zeros2)
    zq = _tc_mid(s_parts, gm, dinv, b1.reshape(1, D), W2, Wc)
    r_parts = _sc_pool(zq, dinv, batp, src_p, dst_p, iota, zeros1)
    out = _tc_final(r_parts, zq, dinv, bat.reshape(1, N), b2.reshape(1, D),
                    Wc, bc.reshape(1, 1))
    return out


# speedup vs baseline: 2.8604x; 1.0004x over previous
import functools

import jax
import jax.numpy as jnp
from jax import lax
from jax.experimental import pallas as pl
from jax.experimental.pallas import tpu as pltpu
from jax.experimental.pallas import tpu_sc as plsc

N = 10000
E = 320000
D = 128
G = 64
NC = 2
NS = 16
NW = NC * NS
CH = 128
CPW = 80
CPH = CPW // 2
EPW = CPW * CH
E_PAD = NW * EPW
NT = 10112
RPT = NT // NS

_mesh = plsc.VectorSubcoreMesh(core_axis_name="c", subcore_axis_name="s")


@functools.partial(
    pl.kernel,
    out_type=jax.ShapeDtypeStruct((NC, NT), jnp.float32),
    mesh=_mesh,
    scratch_types=[
        pltpu.VMEM((CPW, CH), jnp.int32),
        pltpu.VMEM((CH,), jnp.float32),
        pltpu.VMEM_SHARED((NT,), jnp.float32),
    ],
)
def _sc_deg(dst3_hbm, zeros1_hbm, deg_out, dst_v, ones_v, deg_sh):
    c = lax.axis_index("c")
    s = lax.axis_index("s")
    w = c * NS + s

    def fill(i, carry):
        ones_v[pl.ds(i * 16, 16)] = jnp.ones((16,), jnp.float32)
        return carry

    lax.fori_loop(0, CH // 16, fill, 0)

    @pl.when(s == 0)
    def _():
        pltpu.sync_copy(zeros1_hbm, deg_sh)

    pltpu.sync_copy(dst3_hbm.at[w], dst_v)
    plsc.subcore_barrier()

    def body(j, carry):
        pltpu.sync_copy(ones_v, deg_sh.at[dst_v.at[j]], add=True)
        return carry

    lax.fori_loop(0, CPW, body, 0)
    plsc.subcore_barrier()

    @pl.when(s == 0)
    def _():
        pltpu.sync_copy(deg_sh, deg_out.at[c])


@functools.partial(
    pl.kernel,
    out_type=jax.ShapeDtypeStruct((NC, NT, D), jnp.float32),
    mesh=_mesh,
    scratch_types=[
        pltpu.VMEM((CPH, CH), jnp.int32),
        pltpu.VMEM((CPH, CH), jnp.int32),
        pltpu.VMEM((2, CH, D), jnp.float32),
        pltpu.VMEM_SHARED((NT, D), jnp.float32),
        pltpu.SemaphoreType.DMA,
        pltpu.SemaphoreType.DMA,
    ],
)
def _sc_scat(gm_hbm, src3_hbm, dst3_hbm, zeros2_hbm, s_out,
             src_v, dst_v, rows_v, s_sh, sem_a, sem_b):
    c = lax.axis_index("c")
    s = lax.axis_index("s")
    w = c * NS + s
    r0 = s * RPT

    pltpu.sync_copy(zeros2_hbm.at[pl.ds(r0, RPT)], s_sh.at[pl.ds(r0, RPT)])
    plsc.subcore_barrier()

    def gather(j, buf, sem):
        pltpu.async_copy(gm_hbm.at[src_v.at[j]], rows_v.at[buf], sem)

    def gwait(buf, sem):
        pltpu.make_async_copy(gm_hbm.at[src_v.at[0]], rows_v.at[buf], sem).wait()

    def scat(j, buf):
        pltpu.sync_copy(rows_v.at[buf], s_sh.at[dst_v.at[j]], add=True)

    for h in range(2):
        pltpu.sync_copy(src3_hbm.at[w, pl.ds(h * CPH, CPH)], src_v)
        pltpu.sync_copy(dst3_hbm.at[w, pl.ds(h * CPH, CPH)], dst_v)
        gather(0, 0, sem_a)

        def body(k, carry):
            j0 = 2 * k
            gwait(0, sem_a)
            gather(j0 + 1, 1, sem_b)
            scat(j0, 0)

            @pl.when(j0 + 2 < CPH)
            def _():
                gather(j0 + 2, 0, sem_a)

            gwait(1, sem_b)
            scat(j0 + 1, 1)
            return carry

        lax.fori_loop(0, CPH // 2, body, 0)
    plsc.subcore_barrier()
    pltpu.sync_copy(s_sh.at[pl.ds(r0, RPT)], s_out.at[c, pl.ds(r0, RPT)])


@functools.partial(
    pl.kernel,
    out_type=jax.ShapeDtypeStruct((NC, CH), jnp.float32),
    mesh=_mesh,
    scratch_types=[
        pltpu.VMEM((NT,), jnp.float32),
        pltpu.VMEM((NT,), jnp.float32),
        pltpu.VMEM((NT,), jnp.int32),
        pltpu.VMEM((EPW,), jnp.int32),
        pltpu.VMEM((EPW,), jnp.int32),
        pltpu.VMEM((16 * CH,), jnp.float32),
        pltpu.VMEM((CH,), jnp.float32),
        pltpu.VMEM((1, CH), jnp.int32),
        pltpu.VMEM_SHARED((CH,), jnp.float32),
    ],
    compiler_params=pltpu.CompilerParams(needs_layout_passes=False),
)
def _sc_pool(zq_hbm, dq_hbm, batp_hbm, srcp_hbm, dstp_hbm, iota_hbm,
             zeros1_hbm, r_out,
             zq_v, dq_v, bat_v, src_v, dst_v, r_v, r2_v, io_v, r_sh):
    c = lax.axis_index("c")
    s = lax.axis_index("s")
    w = c * NS + s

    pltpu.sync_copy(zq_hbm.at[0], zq_v)
    pltpu.sync_copy(dq_hbm.at[0], dq_v)
    pltpu.sync_copy(batp_hbm, bat_v)
    pltpu.sync_copy(srcp_hbm.at[pl.ds(w * EPW, EPW)], src_v)
    pltpu.sync_copy(dstp_hbm.at[pl.ds(w * EPW, EPW)], dst_v)
    pltpu.sync_copy(iota_hbm, io_v)

    @pl.when(s == 0)
    def _():
        pltpu.sync_copy(zeros1_hbm.at[pl.ds(0, CH)], r_sh)

    def zero(i, carry):
        r_v[pl.ds(i * 16, 16)] = jnp.zeros((16,), jnp.float32)
        return carry

    lax.fori_loop(0, 16 * CH // 16, zero, 0)
    plsc.subcore_barrier()

    lane_off = jnp.arange(16, dtype=jnp.int32) * CH

    def body(e, carry):
        si = src_v[pl.ds(e * 16, 16)]
        di = dst_v[pl.ds(e * 16, 16)]
        val = plsc.load_gather(zq_v, [si]) * plsc.load_gather(dq_v, [di])
        gi = plsc.load_gather(bat_v, [di])
        plsc.addupdate_scatter(r_v, [gi + lane_off], val)
        return carry

    lax.fori_loop(0, EPW // 16, body, 0)

    for cc in range(CH // 16):
        acc = jnp.zeros((16,), jnp.float32)
        for l in range(16):
            acc = acc + r_v[pl.ds(l * CH + cc * 16, 16)]
        r2_v[pl.ds(cc * 16, 16)] = acc

    pltpu.sync_copy(r2_v, r_sh.at[io_v.at[0]], add=True)
    plsc.subcore_barrier()

    @pl.when(s == 0)
    def _():
        pltpu.sync_copy(r_sh, r_out.at[c])


def _tc_mm_body(x_ref, w1_ref, h_ref):
    h_ref[...] = jnp.dot(x_ref[...], w1_ref[...],
                         preferred_element_type=jnp.float32)


def _tc_mm(x, W1):
    return pl.pallas_call(
        _tc_mm_body,
        out_shape=jax.ShapeDtypeStruct((N, D), jnp.float32),
    )(x, W1)


def _tc_prep_body(h_ref, degp_ref, gm_ref, dinv_ref):
    deg = 1.0 + degp_ref[0:1, :] + degp_ref[1:2, :]
    dinv = lax.rsqrt(deg)
    col = lax.broadcasted_iota(jnp.int32, (1, NT), 1)
    dinv = jnp.where(col < N, dinv, 0.0)
    dinv_ref[...] = dinv
    dcol = dinv[0, :N].reshape(N, 1)
    gm_ref[...] = h_ref[...] * dcol


def _tc_prep(h, deg_parts):
    return pl.pallas_call(
        _tc_prep_body,
        out_shape=(
            jax.ShapeDtypeStruct((N, D), jnp.float32),
            jax.ShapeDtypeStruct((1, NT), jnp.float32),
        ),
    )(h, deg_parts)


def _tc_mid_body(sp_ref, gm_ref, dinv_ref, b1_ref, w2_ref, wc_ref, zq_ref):
    S = sp_ref[0, 0:N, :] + sp_ref[1, 0:N, :]
    dcol = dinv_ref[0, 0:N].reshape(N, 1)
    h1 = dcol * (S + gm_ref[...]) + b1_ref[0, :][None, :]
    h1r = jnp.maximum(h1, 0.0)
    v = jnp.dot(w2_ref[...], wc_ref[...], preferred_element_type=jnp.float32)
    z = jnp.dot(h1r, v, preferred_element_type=jnp.float32)
    zq = dcol * z
    zq_ref[:, pl.ds(0, N)] = zq.reshape(1, N)
    zq_ref[:, pl.ds(N, NT - N)] = jnp.zeros((1, NT - N), jnp.float32)


def _tc_mid(s_parts, gm, dinv, b1, W2, Wc):
    return pl.pallas_call(
        _tc_mid_body,
        out_shape=jax.ShapeDtypeStruct((1, NT), jnp.float32),
    )(s_parts, gm, dinv, b1, W2, Wc)


def _tc_self_body(zq_ref, dinv_ref, bat_ref, rs_ref):
    zqv = zq_ref[0, 0:N]
    dv = dinv_ref[0, 0:N]
    contrib = zqv * dv
    bat = bat_ref[0, :]
    gid = lax.broadcasted_iota(jnp.int32, (G, N), 0)
    m = (gid == bat[None, :]).astype(jnp.float32)
    rs_ref[0, :] = jnp.sum(m * contrib[None, :], axis=1)
    rs_ref[1, :] = jnp.sum(m, axis=1)


def _tc_self(zq, dinv, batch2):
    return pl.pallas_call(
        _tc_self_body,
        out_shape=jax.ShapeDtypeStruct((2, G), jnp.float32),
    )(zq, dinv, batch2)


def _tc_final_body(rp_ref, rs_ref, b2_ref, wc_ref, bc_ref, out_ref):
    r_edge = rp_ref[0, 0:G] + rp_ref[1, 0:G]
    r_self = rs_ref[0, :]
    counts = rs_ref[1, :]
    cm = jnp.maximum(counts, 1.0)
    bw = jnp.dot(b2_ref[...], wc_ref[...],
                 preferred_element_type=jnp.float32)[0, 0]
    outv = (r_edge + r_self) / cm + (counts / cm) * bw + bc_ref[0, 0]
    out_ref[...] = outv.reshape(G, 1)


def _tc_final(r_parts, r_self, b2, Wc, bc):
    return pl.pallas_call(
        _tc_final_body,
        out_shape=jax.ShapeDtypeStruct((G, 1), jnp.float32),
    )(r_parts, r_self, b2, Wc, bc)


def kernel(x, edge_index, batch, W1, b1, W2, b2, Wc, bc):
    src = edge_index[0].astype(jnp.int32)
    dst = edge_index[1].astype(jnp.int32)
    bat = batch.astype(jnp.int32)

    pad = E_PAD - E
    pad_i = jnp.arange(pad, dtype=jnp.int32)
    src_p = jnp.concatenate([src, pad_i % N])
    dst_p = jnp.concatenate([dst, N + pad_i % (NT - N)])
    src3 = src_p.reshape(NW, CPW, CH)
    dst3 = dst_p.reshape(NW, CPW, CH)
    src4 = src_p.reshape(NW, CPW // 2, 2, CH)
    dst4 = dst_p.reshape(NW, CPW // 2, 2, CH)
    batp = jnp.concatenate([bat, jnp.full((NT - N,), G, jnp.int32)])
    zeros1 = jnp.zeros((NT,), jnp.float32)
    zeros2 = jnp.zeros((NT, D), jnp.float32)
    iota = jnp.arange(CH, dtype=jnp.int32).reshape(1, CH)

    h = _tc_mm(x, W1)
    deg_parts = _sc_deg(dst3, zeros1)
    gm, dinv = _tc_prep(h, deg_parts)
    s_parts = _sc_scat(gm, src3, dst3, zeros2)
    zq = _tc_mid(s_parts, gm, dinv, b1.reshape(1, D), W2, Wc)
    r_parts = _sc_pool(zq, dinv, batp, src_p, dst_p, iota, zeros1)
    r_self = _tc_self(zq, dinv, bat.reshape(1, N))
    out = _tc_final(r_parts, r_self, b2.reshape(1, D), Wc, bc.reshape(1, 1))
    return out
